# Initial kernel scaffold; baseline (speedup 1.0000x reference)
import jax
import jax.numpy as jnp
from jax.experimental import pallas as pl


def kernel(x, Wg, W1, W2):
    raise NotImplementedError("write your pallas kernel here")


# speedup vs baseline: 1.6444x; 1.6444x over previous
import functools

import jax
import jax.numpy as jnp
from jax.experimental import pallas as pl
from jax.experimental.pallas import tpu as pltpu

B, T, D = 32, 8, 1024
E, FF = 16, 2048
N = B * T


def _moe_kernel(h_ref, wg_ref, w1_ref, w2_ref, y_ref, wts_ref):
    e = pl.program_id(0)

    @pl.when(e == 0)
    def _gating():
        h = h_ref[...]
        wg = wg_ref[...]
        logits = jax.lax.dot_general(
            wg, h, (((1,), (1,)), ((), ())),
            preferred_element_type=jnp.float32)
        m1 = jnp.max(logits, axis=0, keepdims=True)
        eidx = jax.lax.broadcasted_iota(jnp.int32, (E, N), 0)
        big = jnp.int32(E)
        first_max = jnp.min(
            jnp.where(logits == m1, eidx, big), axis=0, keepdims=True)
        neg_inf = jnp.float32(-jnp.inf)
        masked = jnp.where(eidx == first_max, neg_inf, logits)
        m2 = jnp.max(masked, axis=0, keepdims=True)
        sel = logits >= m2
        w = jnp.where(sel, jnp.exp(logits - m1), 0.0)
        w = w / jnp.sum(w, axis=0, keepdims=True)
        wts_ref[...] = w

    h = h_ref[...]
    w1 = w1_ref[0]
    w2 = w2_ref[0]
    hid = jax.lax.dot_general(
        h, w1, (((1,), (1,)), ((), ())), preferred_element_type=jnp.float32)
    hid = hid * jax.nn.sigmoid(hid)
    out = jax.lax.dot_general(
        hid, w2, (((1,), (1,)), ((), ())), preferred_element_type=jnp.float32)
    w = wts_ref[e, :].reshape(N, 1)

    @pl.when(e == 0)
    def _init():
        y_ref[...] = w * out

    @pl.when(e > 0)
    def _acc():
        y_ref[...] = y_ref[...] + w * out


def kernel(x, Wg, W1, W2):
    h = x.reshape(N, D)
    y = pl.pallas_call(
        _moe_kernel,
        grid=(E,),
        in_specs=[
            pl.BlockSpec((N, D), lambda e: (0, 0)),
            pl.BlockSpec((E, D), lambda e: (0, 0)),
            pl.BlockSpec((1, FF, D), lambda e: (e, 0, 0)),
            pl.BlockSpec((1, D, FF), lambda e: (e, 0, 0)),
        ],
        out_specs=pl.BlockSpec((N, D), lambda e: (0, 0)),
        out_shape=jax.ShapeDtypeStruct((N, D), jnp.float32),
        scratch_shapes=[pltpu.VMEM((E, N), jnp.float32)],
        compiler_params=pltpu.CompilerParams(
            dimension_semantics=("arbitrary",)),
    )(h, Wg, W1, W2)
    return y.reshape(B, T, D)
